# baseline (speedup 1.0000x reference)
import functools

import jax
import jax.numpy as jnp
from jax import lax
from jax.experimental import pallas as pl
from jax.experimental.pallas import tpu as pltpu
from jax.experimental.pallas import tpu_sc as plsc

_NC, _NS = 2, 16
_NW = _NC * _NS
_EMB = 32
_FW = 4 * _EMB
_CHUNK = 128
_GCH = 256


def _fuse_body(a_r, b_r, c_r, d_r, f_r):
    stacked = jnp.concatenate(
        [a_r[...], b_r[...], c_r[...], d_r[...]], axis=0)
    f_r[...] = stacked.T.astype(jnp.bfloat16)


def _tc_fuse_tables(a, b, c, d):
    n = a.shape[1]
    blk = 16384
    grid = pl.cdiv(n, blk)
    in_spec = pl.BlockSpec((_EMB, blk), lambda i: (0, i))
    return pl.pallas_call(
        _fuse_body,
        grid=(grid,),
        in_specs=[in_spec] * 4,
        out_specs=pl.BlockSpec((blk, _FW), lambda i: (i, 0)),
        out_shape=jax.ShapeDtypeStruct((n, _FW), jnp.bfloat16),
        compiler_params=pltpu.CompilerParams(vmem_limit_bytes=100 * 2**20),
    )(a, b, c, d)


def _sc_gather(u2, v2, n2, fused, B, BN):
    cb = B // _NW
    cn = BN // _NW
    mesh = plsc.VectorSubcoreMesh(core_axis_name="c", subcore_axis_name="s",
                                  num_cores=_NC, num_subcores=_NS)
    out_type = tuple(
        jax.ShapeDtypeStruct((sz, _FW), jnp.bfloat16) for sz in (B, B, BN)
    )
    scratch = [
        pltpu.VMEM((cb // _CHUNK, _CHUNK), jnp.int32),
        pltpu.VMEM((cb // _CHUNK, _CHUNK), jnp.int32),
        pltpu.VMEM((cn // _CHUNK, _CHUNK), jnp.int32),
        pltpu.VMEM((_GCH, _FW), jnp.bfloat16),
        pltpu.VMEM((_GCH, _FW), jnp.bfloat16),
        pltpu.VMEM((_GCH, _FW), jnp.bfloat16),
        pltpu.SemaphoreType.DMA,
        pltpu.SemaphoreType.DMA,
        pltpu.SemaphoreType.DMA,
        pltpu.SemaphoreType.DMA,
        pltpu.SemaphoreType.DMA,
    ]

    @functools.partial(pl.kernel, mesh=mesh, out_type=out_type,
                       scratch_types=scratch,
                       compiler_params=pltpu.CompilerParams(
                           use_tc_tiling_on_sc=False))
    def k(u_h, v_h, n_h, f_h, ru_o, rv_o, rn_o,
          ui_v, vi_v, ni_v, buf_a, buf_b, buf_c,
          sg_a, sg_b, sw_a, sw_b, sw_c):
        w = lax.axis_index("s") * _NC + lax.axis_index("c")
        pltpu.sync_copy(u_h.at[w], ui_v)
        pltpu.sync_copy(v_h.at[w], vi_v)
        pltpu.sync_copy(n_h.at[w], ni_v)

        chunks = []
        for ci in range(cb // _GCH):
            chunks.append((ui_v, ci, ru_o, w * cb + ci * _GCH))
        for ci in range(cb // _GCH):
            chunks.append((vi_v, ci, rv_o, w * cb + ci * _GCH))
        for ci in range(cn // _GCH):
            chunks.append((ni_v, ci, rn_o, w * cn + ci * _GCH))

        k_per = _GCH // _CHUNK
        bufs = (buf_a, buf_b, buf_c)
        gsems = (sg_a, sg_b)
        wsems = (sw_a, sw_b, sw_c)
        nch = len(chunks)
        gdescs = [None] * nch
        wdescs = [None] * nch

        def fire(t):
            idx_v, ci, _, _ = chunks[t]
            gdescs[t] = [
                pltpu.async_copy(
                    f_h.at[idx_v.at[ci * k_per + i]],
                    bufs[t % 3].at[pl.ds(i * _CHUNK, _CHUNK)],
                    gsems[t % 2])
                for i in range(k_per)
            ]

        def writeback(t):
            _, _, out_h, base = chunks[t]
            for d in gdescs[t]:
                d.wait()
            wdescs[t] = pltpu.async_copy(
                bufs[t % 3], out_h.at[pl.ds(base, _GCH)], wsems[t % 3])

        fire(0)
        for t in range(1, nch):
            if t >= 3:
                wdescs[t - 3].wait()
            fire(t)
            writeback(t - 1)
        writeback(nch - 1)
        wdescs[nch - 2].wait()
        wdescs[nch - 1].wait()

    return k(u2, v2, n2, fused)


def _tc_body(ru_r, rv_r, rn_r, w1u_r, w1v_r, w2p_r, wp1m_r, wones_r,
             wp2m_r, b1_r, b2p_r, bp_r, p_r):
    w1v = w1v_r[...]
    w2p = w2p_r[...]
    wp1m = wp1m_r[...]
    b1 = b1_r[...]
    b2p = b2p_r[...]
    ru = ru_r[...].astype(jnp.float32)
    au = jnp.dot(ru, w1u_r[...], preferred_element_type=jnp.float32)

    p = jnp.zeros(p_r.shape, jnp.float32) + bp_r[0, 0]
    heads = [rv_r[...].astype(jnp.float32)] + [
        rn_r[j].astype(jnp.float32) for j in range(4)]
    for h, rows in enumerate(heads):
        h1 = jnp.maximum(
            au + jnp.dot(rows, w1v, preferred_element_type=jnp.float32)
            + b1, 0.0)
        h2p = jnp.maximum(
            jnp.dot(h1, w2p, preferred_element_type=jnp.float32) + b2p, 0.0)
        gmf_w = ru * jnp.roll(rows, -_EMB, axis=1) * wp1m
        p = (p
             + jnp.dot(gmf_w, wones_r[h],
                       preferred_element_type=jnp.float32)
             + jnp.dot(h2p, wp2m_r[h], preferred_element_type=jnp.float32))
    p_r[...] = p


def _tc_dense(ru, rv, rn, w1u, w1v, w2p, wp1m, wones, wp2m, b1, b2p, bp):
    B = ru.shape[0]
    blk = 2048
    nb = B // blk
    row_spec = pl.BlockSpec((blk, _FW), lambda i: (i, 0))
    neg_spec = pl.BlockSpec((4, blk, _FW), lambda i: (0, i, 0))

    def full(a):
        return pl.BlockSpec(a.shape, lambda i: (0,) * a.ndim)

    return pl.pallas_call(
        _tc_body,
        grid=(nb,),
        in_specs=[row_spec, row_spec, neg_spec,
                  full(w1u), full(w1v), full(w2p), full(wp1m), full(wones),
                  full(wp2m), full(b1), full(b2p), full(bp)],
        out_specs=pl.BlockSpec((blk, 8), lambda i: (i, 0)),
        out_shape=jax.ShapeDtypeStruct((B, 8), jnp.float32),
    )(ru, rv, rn.reshape(4, B, _FW), w1u, w1v, w2p, wp1m, wones, wp2m,
      b1, b2p, bp)


def kernel(u, v, n, gmf_u_emb, gmf_v_emb, u_emb, v_emb, W1, b1, W2, b2, Wp, bp):
    B = u.shape[0]
    nneg = n.shape[1]
    BN = B * nneg
    cb = B // _NW
    cn = BN // _NW
    u2 = u.astype(jnp.int32).reshape(_NW, cb // _CHUNK, _CHUNK)
    v2 = v.astype(jnp.int32).reshape(_NW, cb // _CHUNK, _CHUNK)
    n2 = n.astype(jnp.int32).T.reshape(_NW, cn // _CHUNK, _CHUNK)

    fused = _tc_fuse_tables(gmf_u_emb.T, gmf_v_emb.T, u_emb.T, v_emb.T)
    ru, rv, rn = _sc_gather(u2, v2, n2, fused, B, BN)

    w1u = jnp.zeros((_FW, _EMB), jnp.float32).at[2 * _EMB:3 * _EMB].set(
        W1[:_EMB])
    w1v = jnp.zeros((_FW, _EMB), jnp.float32).at[3 * _EMB:].set(W1[_EMB:])
    w2p = jnp.zeros((_EMB, _FW), jnp.float32).at[:, :16].set(W2)
    wp1m = jnp.zeros((1, _FW), jnp.float32).at[0, :_EMB].set(Wp[:_EMB, 0])
    wones = jnp.zeros((5, _FW, 8), jnp.float32).at[
        jnp.arange(5)[:, None], jnp.arange(_EMB)[None, :],
        jnp.arange(5)[:, None]].set(1.0)
    wp2m = jnp.zeros((5, _FW, 8), jnp.float32).at[
        jnp.arange(5)[:, None], jnp.arange(16)[None, :],
        jnp.arange(5)[:, None]].set(Wp[_EMB:, 0][None, :])
    b2p = jnp.zeros((1, _FW), jnp.float32).at[0, :16].set(b2)
    p = _tc_dense(ru, rv, rn, w1u, w1v, w2p, wp1m, wones, wp2m,
                  b1.reshape(1, _EMB), b2p, bp.reshape(1, 1))
    return (p[:, 0], p[:, 1:5].reshape(-1))


# speedup vs baseline: 2.9937x; 2.9937x over previous
import functools

import jax
import jax.numpy as jnp
from jax import lax
from jax.experimental import pallas as pl
from jax.experimental.pallas import tpu as pltpu
from jax.experimental.pallas import tpu_sc as plsc

_NC, _NS = 2, 16
_NW = _NC * _NS
_EMB = 32
_FW = 4 * _EMB
_CHUNK = 128
_GCH = 256


def _fuse_body(a_r, b_r, c_r, d_r, f_r):
    stacked = jnp.concatenate(
        [a_r[...], b_r[...], c_r[...], d_r[...]], axis=0)
    f_r[...] = stacked.T


def _tc_fuse_tables(a, b, c, d):
    n = a.shape[1]
    blk = 16384
    grid = pl.cdiv(n, blk)
    in_spec = pl.BlockSpec((_EMB, blk), lambda i: (0, i))
    return pl.pallas_call(
        _fuse_body,
        grid=(grid,),
        in_specs=[in_spec] * 4,
        out_specs=pl.BlockSpec((blk, _FW), lambda i: (i, 0)),
        out_shape=jax.ShapeDtypeStruct((n, _FW), jnp.float32),
        compiler_params=pltpu.CompilerParams(vmem_limit_bytes=100 * 2**20),
    )(a, b, c, d)


def _sc_gather(u2, v2, n2, fused, B, BN):
    cb = B // _NW
    cn = BN // _NW
    mesh = plsc.VectorSubcoreMesh(core_axis_name="c", subcore_axis_name="s",
                                  num_cores=_NC, num_subcores=_NS)
    out_type = tuple(
        jax.ShapeDtypeStruct((sz, _FW), jnp.float32) for sz in (B, B, BN)
    )
    scratch = [
        pltpu.VMEM((cb // _CHUNK, _CHUNK), jnp.int32),
        pltpu.VMEM((cb // _CHUNK, _CHUNK), jnp.int32),
        pltpu.VMEM((cn // _CHUNK, _CHUNK), jnp.int32),
        pltpu.VMEM((_GCH, _FW), jnp.float32),
        pltpu.VMEM((_GCH, _FW), jnp.float32),
        pltpu.VMEM((_GCH, _FW), jnp.float32),
        pltpu.SemaphoreType.DMA,
        pltpu.SemaphoreType.DMA,
        pltpu.SemaphoreType.DMA,
        pltpu.SemaphoreType.DMA,
        pltpu.SemaphoreType.DMA,
    ]

    @functools.partial(pl.kernel, mesh=mesh, out_type=out_type,
                       scratch_types=scratch,
                       compiler_params=pltpu.CompilerParams(
                           use_tc_tiling_on_sc=False))
    def k(u_h, v_h, n_h, f_h, ru_o, rv_o, rn_o,
          ui_v, vi_v, ni_v, buf_a, buf_b, buf_c,
          sg_a, sg_b, sw_a, sw_b, sw_c):
        w = lax.axis_index("s") * _NC + lax.axis_index("c")
        pltpu.sync_copy(u_h.at[w], ui_v)
        pltpu.sync_copy(v_h.at[w], vi_v)
        pltpu.sync_copy(n_h.at[w], ni_v)

        chunks = []
        for ci in range(cb // _GCH):
            chunks.append((ui_v, ci, ru_o, w * cb + ci * _GCH))
        for ci in range(cb // _GCH):
            chunks.append((vi_v, ci, rv_o, w * cb + ci * _GCH))
        for ci in range(cn // _GCH):
            chunks.append((ni_v, ci, rn_o, w * cn + ci * _GCH))

        k_per = _GCH // _CHUNK
        bufs = (buf_a, buf_b, buf_c)
        gsems = (sg_a, sg_b)
        wsems = (sw_a, sw_b, sw_c)
        nch = len(chunks)
        gdescs = [None] * nch
        wdescs = [None] * nch

        def fire(t):
            idx_v, ci, _, _ = chunks[t]
            gdescs[t] = [
                pltpu.async_copy(
                    f_h.at[idx_v.at[ci * k_per + i]],
                    bufs[t % 3].at[pl.ds(i * _CHUNK, _CHUNK)],
                    gsems[t % 2])
                for i in range(k_per)
            ]

        def writeback(t):
            _, _, out_h, base = chunks[t]
            for d in gdescs[t]:
                d.wait()
            wdescs[t] = pltpu.async_copy(
                bufs[t % 3], out_h.at[pl.ds(base, _GCH)], wsems[t % 3])

        fire(0)
        for t in range(1, nch):
            if t >= 3:
                wdescs[t - 3].wait()
            fire(t)
            writeback(t - 1)
        writeback(nch - 1)
        wdescs[nch - 2].wait()
        wdescs[nch - 1].wait()

    return k(u2, v2, n2, fused)


def _tc_body(ru_r, rv_r, rn_r, w1u_r, w1v_r, w2p_r, wp1m_r, wones_r,
             wp2m_r, b1_r, b2p_r, bp_r, p_r):
    w1v = w1v_r[...]
    w2p = w2p_r[...]
    wp1m = wp1m_r[...]
    b1 = b1_r[...]
    b2p = b2p_r[...]
    ru = ru_r[...]
    au = jnp.dot(ru, w1u_r[...], preferred_element_type=jnp.float32)

    p = jnp.zeros(p_r.shape, jnp.float32) + bp_r[0, 0]
    heads = [rv_r[...]] + [rn_r[j] for j in range(4)]
    for h, rows in enumerate(heads):
        h1 = jnp.maximum(
            au + jnp.dot(rows, w1v, preferred_element_type=jnp.float32)
            + b1, 0.0)
        h2p = jnp.maximum(
            jnp.dot(h1, w2p, preferred_element_type=jnp.float32) + b2p, 0.0)
        gmf_w = ru * jnp.roll(rows, -_EMB, axis=1) * wp1m
        p = (p
             + jnp.dot(gmf_w, wones_r[h],
                       preferred_element_type=jnp.float32)
             + jnp.dot(h2p, wp2m_r[h], preferred_element_type=jnp.float32))
    p_r[...] = p


def _tc_dense(ru, rv, rn, w1u, w1v, w2p, wp1m, wones, wp2m, b1, b2p, bp):
    B = ru.shape[0]
    blk = 2048
    nb = B // blk
    row_spec = pl.BlockSpec((blk, _FW), lambda i: (i, 0))
    neg_spec = pl.BlockSpec((4, blk, _FW), lambda i: (0, i, 0))

    def full(a):
        return pl.BlockSpec(a.shape, lambda i: (0,) * a.ndim)

    return pl.pallas_call(
        _tc_body,
        grid=(nb,),
        in_specs=[row_spec, row_spec, neg_spec,
                  full(w1u), full(w1v), full(w2p), full(wp1m), full(wones),
                  full(wp2m), full(b1), full(b2p), full(bp)],
        out_specs=pl.BlockSpec((blk, 8), lambda i: (i, 0)),
        out_shape=jax.ShapeDtypeStruct((B, 8), jnp.float32),
    )(ru, rv, rn.reshape(4, B, _FW), w1u, w1v, w2p, wp1m, wones, wp2m,
      b1, b2p, bp)


def kernel(u, v, n, gmf_u_emb, gmf_v_emb, u_emb, v_emb, W1, b1, W2, b2, Wp, bp):
    B = u.shape[0]
    nneg = n.shape[1]
    BN = B * nneg
    cb = B // _NW
    cn = BN // _NW
    u2 = u.astype(jnp.int32).reshape(_NW, cb // _CHUNK, _CHUNK)
    v2 = v.astype(jnp.int32).reshape(_NW, cb // _CHUNK, _CHUNK)
    n2 = n.astype(jnp.int32).T.reshape(_NW, cn // _CHUNK, _CHUNK)

    fused = _tc_fuse_tables(gmf_u_emb.T, gmf_v_emb.T, u_emb.T, v_emb.T)
    ru, rv, rn = _sc_gather(u2, v2, n2, fused, B, BN)

    w1u = jnp.zeros((_FW, _EMB), jnp.float32).at[2 * _EMB:3 * _EMB].set(
        W1[:_EMB])
    w1v = jnp.zeros((_FW, _EMB), jnp.float32).at[3 * _EMB:].set(W1[_EMB:])
    w2p = jnp.zeros((_EMB, _FW), jnp.float32).at[:, :16].set(W2)
    wp1m = jnp.zeros((1, _FW), jnp.float32).at[0, :_EMB].set(Wp[:_EMB, 0])
    wones = jnp.zeros((5, _FW, 8), jnp.float32).at[
        jnp.arange(5)[:, None], jnp.arange(_EMB)[None, :],
        jnp.arange(5)[:, None]].set(1.0)
    wp2m = jnp.zeros((5, _FW, 8), jnp.float32).at[
        jnp.arange(5)[:, None], jnp.arange(16)[None, :],
        jnp.arange(5)[:, None]].set(Wp[_EMB:, 0][None, :])
    b2p = jnp.zeros((1, _FW), jnp.float32).at[0, :16].set(b2)
    p = _tc_dense(ru, rv, rn, w1u, w1v, w2p, wp1m, wones, wp2m,
                  b1.reshape(1, _EMB), b2p, bp.reshape(1, 1))
    return (p[:, 0], p[:, 1:5].reshape(-1))


# speedup vs baseline: 2.9962x; 1.0008x over previous
import functools

import jax
import jax.numpy as jnp
from jax import lax
from jax.experimental import pallas as pl
from jax.experimental.pallas import tpu as pltpu
from jax.experimental.pallas import tpu_sc as plsc

_NC, _NS = 2, 16
_NW = _NC * _NS
_EMB = 32
_FW = 4 * _EMB
_CHUNK = 128
_GCH = 256


def _fuse_body(a_r, b_r, c_r, d_r, f_r):
    stacked = jnp.concatenate(
        [a_r[...], b_r[...], c_r[...], d_r[...]], axis=0)
    f_r[...] = stacked.T


def _tc_fuse_tables(a, b, c, d):
    n = a.shape[1]
    blk = 24576
    grid = pl.cdiv(n, blk)
    in_spec = pl.BlockSpec((_EMB, blk), lambda i: (0, i))
    return pl.pallas_call(
        _fuse_body,
        grid=(grid,),
        in_specs=[in_spec] * 4,
        out_specs=pl.BlockSpec((blk, _FW), lambda i: (i, 0)),
        out_shape=jax.ShapeDtypeStruct((n, _FW), jnp.float32),
        compiler_params=pltpu.CompilerParams(vmem_limit_bytes=100 * 2**20),
    )(a, b, c, d)


def _sc_gather(u2, v2, n2, fused, B, BN):
    cb = B // _NW
    cn = BN // _NW
    mesh = plsc.VectorSubcoreMesh(core_axis_name="c", subcore_axis_name="s",
                                  num_cores=_NC, num_subcores=_NS)
    out_type = tuple(
        jax.ShapeDtypeStruct((sz, _FW), jnp.float32) for sz in (B, B, BN)
    )
    scratch = [
        pltpu.VMEM((cb // _CHUNK, _CHUNK), jnp.int32),
        pltpu.VMEM((cb // _CHUNK, _CHUNK), jnp.int32),
        pltpu.VMEM((cn // _CHUNK, _CHUNK), jnp.int32),
        pltpu.VMEM((_GCH, _FW), jnp.float32),
        pltpu.VMEM((_GCH, _FW), jnp.float32),
        pltpu.VMEM((_GCH, _FW), jnp.float32),
        pltpu.SemaphoreType.DMA,
        pltpu.SemaphoreType.DMA,
        pltpu.SemaphoreType.DMA,
        pltpu.SemaphoreType.DMA,
        pltpu.SemaphoreType.DMA,
    ]

    @functools.partial(pl.kernel, mesh=mesh, out_type=out_type,
                       scratch_types=scratch,
                       compiler_params=pltpu.CompilerParams(
                           use_tc_tiling_on_sc=False))
    def k(u_h, v_h, n_h, f_h, ru_o, rv_o, rn_o,
          ui_v, vi_v, ni_v, buf_a, buf_b, buf_c,
          sg_a, sg_b, sw_a, sw_b, sw_c):
        w = lax.axis_index("s") * _NC + lax.axis_index("c")
        pltpu.sync_copy(u_h.at[w], ui_v)
        pltpu.sync_copy(v_h.at[w], vi_v)
        pltpu.sync_copy(n_h.at[w], ni_v)

        chunks = []
        for ci in range(cb // _GCH):
            chunks.append((ui_v, ci, ru_o, w * cb + ci * _GCH))
        for ci in range(cb // _GCH):
            chunks.append((vi_v, ci, rv_o, w * cb + ci * _GCH))
        for ci in range(cn // _GCH):
            chunks.append((ni_v, ci, rn_o, w * cn + ci * _GCH))

        k_per = _GCH // _CHUNK
        bufs = (buf_a, buf_b, buf_c)
        gsems = (sg_a, sg_b)
        wsems = (sw_a, sw_b, sw_c)
        nch = len(chunks)
        gdescs = [None] * nch
        wdescs = [None] * nch

        def fire(t):
            idx_v, ci, _, _ = chunks[t]
            gdescs[t] = [
                pltpu.async_copy(
                    f_h.at[idx_v.at[ci * k_per + i]],
                    bufs[t % 3].at[pl.ds(i * _CHUNK, _CHUNK)],
                    gsems[t % 2])
                for i in range(k_per)
            ]

        def writeback(t):
            _, _, out_h, base = chunks[t]
            for d in gdescs[t]:
                d.wait()
            wdescs[t] = pltpu.async_copy(
                bufs[t % 3], out_h.at[pl.ds(base, _GCH)], wsems[t % 3])

        fire(0)
        for t in range(1, nch):
            if t >= 3:
                wdescs[t - 3].wait()
            fire(t)
            writeback(t - 1)
        writeback(nch - 1)
        wdescs[nch - 2].wait()
        wdescs[nch - 1].wait()

    return k(u2, v2, n2, fused)


def _tc_body(ru_r, rv_r, rn_r, w1u_r, w1v_r, w2p_r, wp1m_r, wones_r,
             wp2m_r, b1_r, b2p_r, bp_r, p_r):
    w1v = w1v_r[...]
    w2p = w2p_r[...]
    wp1m = wp1m_r[...]
    b1 = b1_r[...]
    b2p = b2p_r[...]
    ru = ru_r[...]
    au = jnp.dot(ru, w1u_r[...], preferred_element_type=jnp.float32)

    p = jnp.zeros(p_r.shape, jnp.float32) + bp_r[0, 0]
    heads = [rv_r[...]] + [rn_r[j] for j in range(4)]
    for h, rows in enumerate(heads):
        h1 = jnp.maximum(
            au + jnp.dot(rows, w1v, preferred_element_type=jnp.float32)
            + b1, 0.0)
        h2p = jnp.maximum(
            jnp.dot(h1, w2p, preferred_element_type=jnp.float32) + b2p, 0.0)
        gmf_w = ru * jnp.roll(rows, -_EMB, axis=1) * wp1m
        p = (p
             + jnp.dot(gmf_w, wones_r[h],
                       preferred_element_type=jnp.float32)
             + jnp.dot(h2p, wp2m_r[h], preferred_element_type=jnp.float32))
    p_r[...] = p


def _tc_dense(ru, rv, rn, w1u, w1v, w2p, wp1m, wones, wp2m, b1, b2p, bp):
    B = ru.shape[0]
    blk = 4096
    nb = B // blk
    row_spec = pl.BlockSpec((blk, _FW), lambda i: (i, 0))
    neg_spec = pl.BlockSpec((4, blk, _FW), lambda i: (0, i, 0))

    def full(a):
        return pl.BlockSpec(a.shape, lambda i: (0,) * a.ndim)

    return pl.pallas_call(
        _tc_body,
        grid=(nb,),
        in_specs=[row_spec, row_spec, neg_spec,
                  full(w1u), full(w1v), full(w2p), full(wp1m), full(wones),
                  full(wp2m), full(b1), full(b2p), full(bp)],
        out_specs=pl.BlockSpec((blk, 8), lambda i: (i, 0)),
        out_shape=jax.ShapeDtypeStruct((B, 8), jnp.float32),
    )(ru, rv, rn.reshape(4, B, _FW), w1u, w1v, w2p, wp1m, wones, wp2m,
      b1, b2p, bp)


def kernel(u, v, n, gmf_u_emb, gmf_v_emb, u_emb, v_emb, W1, b1, W2, b2, Wp, bp):
    B = u.shape[0]
    nneg = n.shape[1]
    BN = B * nneg
    cb = B // _NW
    cn = BN // _NW
    u2 = u.astype(jnp.int32).reshape(_NW, cb // _CHUNK, _CHUNK)
    v2 = v.astype(jnp.int32).reshape(_NW, cb // _CHUNK, _CHUNK)
    n2 = n.astype(jnp.int32).T.reshape(_NW, cn // _CHUNK, _CHUNK)

    fused = _tc_fuse_tables(gmf_u_emb.T, gmf_v_emb.T, u_emb.T, v_emb.T)
    ru, rv, rn = _sc_gather(u2, v2, n2, fused, B, BN)

    w1u = jnp.zeros((_FW, _EMB), jnp.float32).at[2 * _EMB:3 * _EMB].set(
        W1[:_EMB])
    w1v = jnp.zeros((_FW, _EMB), jnp.float32).at[3 * _EMB:].set(W1[_EMB:])
    w2p = jnp.zeros((_EMB, _FW), jnp.float32).at[:, :16].set(W2)
    wp1m = jnp.zeros((1, _FW), jnp.float32).at[0, :_EMB].set(Wp[:_EMB, 0])
    wones = jnp.zeros((5, _FW, 8), jnp.float32).at[
        jnp.arange(5)[:, None], jnp.arange(_EMB)[None, :],
        jnp.arange(5)[:, None]].set(1.0)
    wp2m = jnp.zeros((5, _FW, 8), jnp.float32).at[
        jnp.arange(5)[:, None], jnp.arange(16)[None, :],
        jnp.arange(5)[:, None]].set(Wp[_EMB:, 0][None, :])
    b2p = jnp.zeros((1, _FW), jnp.float32).at[0, :16].set(b2)
    p = _tc_dense(ru, rv, rn, w1u, w1v, w2p, wp1m, wones, wp2m,
                  b1.reshape(1, _EMB), b2p, bp.reshape(1, 1))
    return (p[:, 0], p[:, 1:5].reshape(-1))
